# baseline (speedup 1.0000x reference)
import functools

import jax
import jax.numpy as jnp
from jax import lax
from jax.experimental import pallas as pl
from jax.experimental.pallas import tpu as pltpu
from jax.experimental.pallas import tpu_sc as plsc

_NPOINT = 1024
_RADIUS = 0.2
_NSAMPLE = 32
_BIGI = 2**30

_NR = 8
_NL = 1024


_FPS_BB = 4


def _fps_kernel(xs_ref, xyzs_ref, nxyz_ref):
    n_id = (lax.broadcasted_iota(jnp.int32, (_NR, _NL), 0) * _NL
            + lax.broadcasted_iota(jnp.int32, (_NR, _NL), 1))
    xyzv = [(xs_ref[b, 0], xs_ref[b, 1], xs_ref[b, 2])
            for b in range(_FPS_BB)]

    def body(i, st):
        out = []
        for b in range(_FPS_BB):
            dists, far = st[2 * b], st[2 * b + 1]
            cx = xyzs_ref[b, 0, far]
            cy = xyzs_ref[b, 1, far]
            cz = xyzs_ref[b, 2, far]
            nxyz_ref[b, 0, i] = cx
            nxyz_ref[b, 1, i] = cy
            nxyz_ref[b, 2, i] = cz
            x, y, z = xyzv[b]
            dx = x - cx
            dy = y - cy
            dz = z - cz
            d = (dx * dx + dy * dy) + dz * dz
            dists = jnp.minimum(dists, d)
            m = jnp.max(dists)
            far2 = jnp.min(jnp.where(dists == m, n_id, _BIGI)).astype(
                jnp.int32)
            out += [dists, far2]
        return tuple(out)

    dists0 = jnp.full((_NR, _NL), 1e10, dtype=jnp.float32)
    init = []
    for b in range(_FPS_BB):
        init += [dists0, jnp.int32(0)]
    lax.fori_loop(0, _NPOINT, body, tuple(init))


def _cumsum_lanes(x):
    rows, l = x.shape
    s = 1
    while s < l:
        pad = jnp.zeros((rows, s), x.dtype)
        x = x + jnp.concatenate([pad, x[:, :l - s]], axis=1)
        s *= 2
    return x


def _ballq_kernel(xs_ref, nx_ref, idx_ref, idxf_ref, cnt_ref):
    b = pl.program_id(0)
    n_total = _NR * _NL
    r2 = _RADIUS ** 2
    nxyz = nx_ref[0]
    nx = nxyz[:, 0:1]
    ny = nxyz[:, 1:2]
    nz = nxyz[:, 2:3]
    aa = (nx * nx + ny * ny) + nz * nz
    bf = lambda t: t.astype(jnp.bfloat16).astype(jnp.float32)
    nxb, nyb, nzb = bf(nx), bf(ny), bf(nz)
    idx_ref[0, 0] = jnp.full((128, _NSAMPLE), n_total, dtype=jnp.int32)
    cnt_ref[...] = jnp.zeros((128, 1), dtype=jnp.int32)
    sub = 256
    lane = lax.broadcasted_iota(jnp.int32, (1, sub), 1)

    def chunk(c, carry):
        for j in range(_NL // sub):
            cnt = cnt_ref[...]

            @pl.when(jnp.min(cnt) < _NSAMPLE)
            def _(j=j, cnt=cnt):
                sl = pl.ds(j * sub, sub)
                xr = xs_ref[0, 0, pl.ds(c, 1), sl]
                yr = xs_ref[0, 1, pl.ds(c, 1), sl]
                zr = xs_ref[0, 2, pl.ds(c, 1), sl]
                bb = (xr * xr + yr * yr) + zr * zr
                ab = (nxb * bf(xr) + nyb * bf(yr)) + nzb * bf(zr)
                sqd = (aa + bb) - 2.0 * ab
                mask = sqd < r2
                pos = _cumsum_lanes(mask.astype(jnp.int32))
                nrow = c * _NL + j * sub + lane
                slot = cnt + pos - 1
                nb = jnp.where(mask & (slot < _NSAMPLE), nrow, _BIGI)
                cols = [jnp.min(jnp.where(slot == k, nb, _BIGI), axis=1,
                                keepdims=True)
                        for k in range(_NSAMPLE)]
                vals = jnp.concatenate(cols, axis=1)
                idx_ref[0, 0] = jnp.minimum(idx_ref[0, 0], vals)
                cnt_ref[...] = cnt + pos[:, sub - 1:sub]

        return carry

    lax.fori_loop(0, _NR, chunk, 0)

    o = idx_ref[0, 0]
    first = o[:, 0:1]
    o = jnp.where(o == n_total, first, o)
    idx_ref[0, 0] = o
    idxf_ref[0, 0] = o + b * n_total


def _feat_kernel(cat_ref, w_ref, f_ref):
    f_ref[...] = jnp.dot(cat_ref[...], w_ref[...],
                         preferred_element_type=jnp.float32,
                         precision=jax.lax.Precision.HIGHEST)


def _mlp_kernel(g_ref, nx_ref, w0x_ref, b0_ref, w1_ref, b1_ref, w2_ref, b2_ref,
                out_ref):
    hi = jax.lax.Precision.HIGHEST
    g0 = b0_ref[...] - jnp.dot(nx_ref[...], w0x_ref[...],
                               preferred_element_type=jnp.float32, precision=hi)
    gb = jnp.broadcast_to(g0[:, None, :], (128, _NSAMPLE, 32)).reshape(4096, 32)
    h = jnp.maximum(g_ref[...] + gb, 0.0)
    h = jnp.maximum(jnp.dot(h, w1_ref[...],
                            preferred_element_type=jnp.float32, precision=hi)
                    + b1_ref[...], 0.0)
    h = jnp.maximum(jnp.dot(h, w2_ref[...],
                            preferred_element_type=jnp.float32, precision=hi)
                    + b2_ref[...], 0.0)
    out_ref[...] = jnp.max(h.reshape(128, _NSAMPLE, 64), axis=1)


_SC_NC = 2
_SC_NS = 16
_SC_CH = 128


def _sc_gather_kernel(idx_hbm, table_hbm, out_hbm, idx_v, rows_v, sem0, sem1):
    nw = _SC_NC * _SC_NS
    bt = out_hbm.shape[0]
    per_w = bt // nw
    nch = per_w // _SC_CH
    wid = lax.axis_index("s") * _SC_NC + lax.axis_index("c")
    base = wid * per_w
    pltpu.sync_copy(idx_hbm.at[pl.ds(base, per_w)], idx_v)

    def body(g, carry):
        t0 = 2 * g
        t1 = t0 + 1
        h0 = pltpu.async_copy(
            table_hbm.at[idx_v.at[pl.ds(t0 * _SC_CH, _SC_CH)]],
            rows_v.at[0], sem0)
        h1 = pltpu.async_copy(
            table_hbm.at[idx_v.at[pl.ds(t1 * _SC_CH, _SC_CH)]],
            rows_v.at[1], sem1)
        h0.wait()
        pltpu.sync_copy(rows_v.at[0],
                        out_hbm.at[pl.ds(base + t0 * _SC_CH, _SC_CH)])
        h1.wait()
        pltpu.sync_copy(rows_v.at[1],
                        out_hbm.at[pl.ds(base + t1 * _SC_CH, _SC_CH)])
        return carry

    lax.fori_loop(0, nch // 2, body, 0)


def _stage_fps(xs):
    B = xs.shape[0]
    S = _NPOINT
    bb = _FPS_BB
    xyzs = xs.reshape(B, 3, _NR * _NL)
    fps = pl.pallas_call(
        _fps_kernel,
        grid=(B // bb,),
        in_specs=[
            pl.BlockSpec((bb, 3, _NR, _NL), lambda b: (b, 0, 0, 0)),
            pl.BlockSpec((bb, 3, _NR * _NL), lambda b: (b, 0, 0),
                         memory_space=pltpu.SMEM),
        ],
        out_specs=pl.BlockSpec((bb, 3, S), lambda b: (b, 0, 0),
                               memory_space=pltpu.SMEM),
        out_shape=jax.ShapeDtypeStruct((B, 3, S), jnp.float32),
        compiler_params=pltpu.CompilerParams(
            dimension_semantics=("parallel",)),
    )
    return fps(xs, xyzs).transpose(0, 2, 1)


def _stage_ballq(xs, new_xyz):
    B = xs.shape[0]
    S, K = _NPOINT, _NSAMPLE
    i32 = jnp.int32
    sblk = S // 128
    ballq = pl.pallas_call(
        _ballq_kernel,
        grid=(B, sblk),
        in_specs=[
            pl.BlockSpec((1, 3, _NR, _NL), lambda b, s: (b, 0, 0, 0)),
            pl.BlockSpec((1, 128, 3), lambda b, s: (b, s, 0)),
        ],
        out_specs=[
            pl.BlockSpec((1, 1, 128, K), lambda b, s: (b, s, 0, 0)),
            pl.BlockSpec((1, 1, 128, K), lambda b, s: (b, s, 0, 0)),
        ],
        out_shape=[jax.ShapeDtypeStruct((B, sblk, 128, K), i32),
                   jax.ShapeDtypeStruct((B, sblk, 128, K), i32)],
        scratch_shapes=[pltpu.VMEM((128, 1), i32)],
        compiler_params=pltpu.CompilerParams(
            dimension_semantics=("parallel", "parallel")),
    )
    return ballq(xs, new_xyz)


def _stage_feat(catf, w0p):
    rows = catf.shape[0]
    nfb = 4096
    feat = pl.pallas_call(
        _feat_kernel,
        grid=(rows // nfb,),
        in_specs=[pl.BlockSpec((nfb, 32), lambda i: (i, 0)),
                  pl.BlockSpec((32, 32), lambda i: (0, 0))],
        out_specs=pl.BlockSpec((nfb, 32), lambda i: (i, 0)),
        out_shape=jax.ShapeDtypeStruct((rows, 32), jnp.float32),
        compiler_params=pltpu.CompilerParams(
            dimension_semantics=("parallel",)),
    )
    return feat(catf, w0p)


def _stage_gather(idxf, ftab):
    bt = idxf.shape[0]
    f32, i32 = jnp.float32, jnp.int32
    gather = functools.partial(
        pl.kernel,
        mesh=plsc.VectorSubcoreMesh(core_axis_name="c", subcore_axis_name="s"),
        out_type=jax.ShapeDtypeStruct((bt, 32), f32),
        scratch_types=[pltpu.VMEM((bt // (_SC_NC * _SC_NS),), i32),
                       pltpu.VMEM((2, _SC_CH, 32), f32),
                       pltpu.SemaphoreType.DMA,
                       pltpu.SemaphoreType.DMA],
        compiler_params=pltpu.CompilerParams(use_tc_tiling_on_sc=False),
    )(_sc_gather_kernel)
    return gather(idxf, ftab)


def _stage_mlp(grows, nxp, w0x, b0, W1, b1, W2, b2):
    rows = nxp.shape[0]
    K = _NSAMPLE
    mlp = pl.pallas_call(
        _mlp_kernel,
        grid=(rows // 128,),
        in_specs=[
            pl.BlockSpec((128 * K, 32), lambda i: (i, 0)),
            pl.BlockSpec((128, 8), lambda i: (i, 0)),
            pl.BlockSpec((8, 32), lambda i: (0, 0)),
            pl.BlockSpec((1, 32), lambda i: (0, 0)),
            pl.BlockSpec((32, 32), lambda i: (0, 0)),
            pl.BlockSpec((1, 32), lambda i: (0, 0)),
            pl.BlockSpec((32, 64), lambda i: (0, 0)),
            pl.BlockSpec((1, 64), lambda i: (0, 0)),
        ],
        out_specs=pl.BlockSpec((128, 64), lambda i: (i, 0)),
        out_shape=jax.ShapeDtypeStruct((rows, 64), jnp.float32),
        compiler_params=pltpu.CompilerParams(
            dimension_semantics=("parallel",)),
    )
    return mlp(grows, nxp, w0x, b0.reshape(1, 32), W1, b1.reshape(1, 32),
               W2, b2.reshape(1, 64))


def kernel(xyz, points, W0, b0, W1, b1, W2, b2):
    B, N, _ = xyz.shape
    C = points.shape[-1]
    S, K = _NPOINT, _NSAMPLE
    f32 = jnp.float32

    xs = xyz.transpose(0, 2, 1).reshape(B, 3, _NR, _NL)
    new_xyz = _stage_fps(xs)
    idx4, idxf4 = _stage_ballq(xs, new_xyz)
    idx = idx4.reshape(B, S, K)

    cat = jnp.concatenate(
        [xyz, points, jnp.zeros((B, N, 32 - 3 - C), f32)], axis=-1)
    catf = cat.reshape(B * N, 32)
    w0p = jnp.concatenate([W0, jnp.zeros((32 - 3 - C, 32), f32)], axis=0)
    ftab = _stage_feat(catf, w0p)

    grows = _stage_gather(idxf4.reshape(B * S * K), ftab)

    nxp = jnp.concatenate([new_xyz, jnp.zeros((B, S, 5), f32)],
                          axis=-1).reshape(B * S, 8)
    w0x = jnp.concatenate([W0[:3], jnp.zeros((5, 32), f32)], axis=0)
    new_points = _stage_mlp(grows, nxp, w0x, b0, W1, b1, W2, b2).reshape(
        B, S, 64)

    return new_xyz, new_points, idx


# speedup vs baseline: 1.2919x; 1.2919x over previous
import functools

import jax
import jax.numpy as jnp
from jax import lax
from jax.experimental import pallas as pl
from jax.experimental.pallas import tpu as pltpu
from jax.experimental.pallas import tpu_sc as plsc

_NPOINT = 1024
_RADIUS = 0.2
_NSAMPLE = 32
_BIGI = 2**30

_NR = 8
_NL = 1024


_FPS_BB = 4


def _fps_kernel(xs_ref, xyzs_ref, nxyz_ref):
    n_id = (lax.broadcasted_iota(jnp.int32, (_NR, _NL), 0) * _NL
            + lax.broadcasted_iota(jnp.int32, (_NR, _NL), 1))
    xyzv = [(xs_ref[b, 0], xs_ref[b, 1], xs_ref[b, 2])
            for b in range(_FPS_BB)]

    def body(i, st):
        out = []
        for b in range(_FPS_BB):
            dists, far = st[2 * b], st[2 * b + 1]
            cx = xyzs_ref[b, 0, far]
            cy = xyzs_ref[b, 1, far]
            cz = xyzs_ref[b, 2, far]
            nxyz_ref[b, 0, i] = cx
            nxyz_ref[b, 1, i] = cy
            nxyz_ref[b, 2, i] = cz
            x, y, z = xyzv[b]
            dx = x - cx
            dy = y - cy
            dz = z - cz
            d = (dx * dx + dy * dy) + dz * dz
            dists = jnp.minimum(dists, d)
            m = jnp.max(dists)
            far2 = jnp.min(jnp.where(dists == m, n_id, _BIGI)).astype(
                jnp.int32)
            out += [dists, far2]
        return tuple(out)

    dists0 = jnp.full((_NR, _NL), 1e10, dtype=jnp.float32)
    init = []
    for b in range(_FPS_BB):
        init += [dists0, jnp.int32(0)]
    lax.fori_loop(0, _NPOINT, body, tuple(init))


def _cumsum_lanes(x):
    rows, l = x.shape
    s = 1
    while s < l:
        pad = jnp.zeros((rows, s), x.dtype)
        x = x + jnp.concatenate([pad, x[:, :l - s]], axis=1)
        s *= 2
    return x


def _ballq_kernel(xs_ref, nx_ref, idx_ref, idxf_ref, cnt_ref):
    b = pl.program_id(0)
    n_total = _NR * _NL
    r2 = _RADIUS ** 2
    nxyz = nx_ref[0]
    nx = nxyz[:, 0:1]
    ny = nxyz[:, 1:2]
    nz = nxyz[:, 2:3]
    aa = (nx * nx + ny * ny) + nz * nz
    bf = lambda t: t.astype(jnp.bfloat16).astype(jnp.float32)
    nxb, nyb, nzb = bf(nx), bf(ny), bf(nz)
    idx_ref[0, 0] = jnp.full((128, _NSAMPLE), n_total, dtype=jnp.int32)
    cnt_ref[...] = jnp.zeros((128, 1), dtype=jnp.int32)
    sub = 1024
    lane = lax.broadcasted_iota(jnp.int32, (1, sub), 1)

    def chunk(c, carry):
        for j in range(_NL // sub):
            cnt = cnt_ref[...]

            @pl.when(jnp.min(cnt) < _NSAMPLE)
            def _(j=j, cnt=cnt):
                sl = pl.ds(j * sub, sub)
                xr = xs_ref[0, 0, pl.ds(c, 1), sl]
                yr = xs_ref[0, 1, pl.ds(c, 1), sl]
                zr = xs_ref[0, 2, pl.ds(c, 1), sl]
                bb = (xr * xr + yr * yr) + zr * zr
                ab = (nxb * bf(xr) + nyb * bf(yr)) + nzb * bf(zr)
                sqd = (aa + bb) - 2.0 * ab
                mask = sqd < r2
                pos = _cumsum_lanes(mask.astype(jnp.int32))
                nrow = c * _NL + j * sub + lane
                slot = cnt + pos - 1
                nb = jnp.where(mask & (slot < _NSAMPLE), nrow, _BIGI)
                cols = [jnp.min(jnp.where(slot == k, nb, _BIGI), axis=1,
                                keepdims=True)
                        for k in range(_NSAMPLE)]
                vals = jnp.concatenate(cols, axis=1)
                idx_ref[0, 0] = jnp.minimum(idx_ref[0, 0], vals)
                cnt_ref[...] = cnt + pos[:, sub - 1:sub]

        return carry

    lax.fori_loop(0, _NR, chunk, 0)

    o = idx_ref[0, 0]
    first = o[:, 0:1]
    o = jnp.where(o == n_total, first, o)
    idx_ref[0, 0] = o
    idxf_ref[0, 0] = o + b * n_total


def _feat_kernel(cat_ref, w_ref, f_ref):
    f_ref[...] = jnp.dot(cat_ref[...], w_ref[...],
                         preferred_element_type=jnp.float32,
                         precision=jax.lax.Precision.HIGHEST)


def _mlp_kernel(g_ref, nx_ref, w0x_ref, b0_ref, w1_ref, b1_ref, w2_ref, b2_ref,
                out_ref):
    hi = jax.lax.Precision.HIGHEST
    g0 = b0_ref[...] - jnp.dot(nx_ref[...], w0x_ref[...],
                               preferred_element_type=jnp.float32, precision=hi)
    gb = jnp.broadcast_to(g0[:, None, :], (128, _NSAMPLE, 32)).reshape(4096, 32)
    h = jnp.maximum(g_ref[...] + gb, 0.0)
    h = jnp.maximum(jnp.dot(h, w1_ref[...],
                            preferred_element_type=jnp.float32, precision=hi)
                    + b1_ref[...], 0.0)
    h = jnp.maximum(jnp.dot(h, w2_ref[...],
                            preferred_element_type=jnp.float32, precision=hi)
                    + b2_ref[...], 0.0)
    out_ref[...] = jnp.max(h.reshape(128, _NSAMPLE, 64), axis=1)


_SC_NC = 2
_SC_NS = 16
_SC_CH = 128


def _sc_gather_kernel(idx_hbm, table_hbm, out_hbm, idx_v, rows_v, sem0, sem1):
    nw = _SC_NC * _SC_NS
    bt = out_hbm.shape[0]
    per_w = bt // nw
    nch = per_w // _SC_CH
    wid = lax.axis_index("s") * _SC_NC + lax.axis_index("c")
    base = wid * per_w
    pltpu.sync_copy(idx_hbm.at[pl.ds(base, per_w)], idx_v)

    def body(g, carry):
        t0 = 2 * g
        t1 = t0 + 1
        h0 = pltpu.async_copy(
            table_hbm.at[idx_v.at[pl.ds(t0 * _SC_CH, _SC_CH)]],
            rows_v.at[0], sem0)
        h1 = pltpu.async_copy(
            table_hbm.at[idx_v.at[pl.ds(t1 * _SC_CH, _SC_CH)]],
            rows_v.at[1], sem1)
        h0.wait()
        pltpu.sync_copy(rows_v.at[0],
                        out_hbm.at[pl.ds(base + t0 * _SC_CH, _SC_CH)])
        h1.wait()
        pltpu.sync_copy(rows_v.at[1],
                        out_hbm.at[pl.ds(base + t1 * _SC_CH, _SC_CH)])
        return carry

    lax.fori_loop(0, nch // 2, body, 0)


def _stage_fps(xs):
    B = xs.shape[0]
    S = _NPOINT
    bb = _FPS_BB
    xyzs = xs.reshape(B, 3, _NR * _NL)
    fps = pl.pallas_call(
        _fps_kernel,
        grid=(B // bb,),
        in_specs=[
            pl.BlockSpec((bb, 3, _NR, _NL), lambda b: (b, 0, 0, 0)),
            pl.BlockSpec((bb, 3, _NR * _NL), lambda b: (b, 0, 0),
                         memory_space=pltpu.SMEM),
        ],
        out_specs=pl.BlockSpec((bb, 3, S), lambda b: (b, 0, 0),
                               memory_space=pltpu.SMEM),
        out_shape=jax.ShapeDtypeStruct((B, 3, S), jnp.float32),
        compiler_params=pltpu.CompilerParams(
            dimension_semantics=("parallel",)),
    )
    return fps(xs, xyzs).transpose(0, 2, 1)


def _stage_ballq(xs, new_xyz):
    B = xs.shape[0]
    S, K = _NPOINT, _NSAMPLE
    i32 = jnp.int32
    sblk = S // 128
    ballq = pl.pallas_call(
        _ballq_kernel,
        grid=(B, sblk),
        in_specs=[
            pl.BlockSpec((1, 3, _NR, _NL), lambda b, s: (b, 0, 0, 0)),
            pl.BlockSpec((1, 128, 3), lambda b, s: (b, s, 0)),
        ],
        out_specs=[
            pl.BlockSpec((1, 1, 128, K), lambda b, s: (b, s, 0, 0)),
            pl.BlockSpec((1, 1, 128, K), lambda b, s: (b, s, 0, 0)),
        ],
        out_shape=[jax.ShapeDtypeStruct((B, sblk, 128, K), i32),
                   jax.ShapeDtypeStruct((B, sblk, 128, K), i32)],
        scratch_shapes=[pltpu.VMEM((128, 1), i32)],
        compiler_params=pltpu.CompilerParams(
            dimension_semantics=("parallel", "parallel")),
    )
    return ballq(xs, new_xyz)


def _stage_feat(catf, w0p):
    rows = catf.shape[0]
    nfb = 4096
    feat = pl.pallas_call(
        _feat_kernel,
        grid=(rows // nfb,),
        in_specs=[pl.BlockSpec((nfb, 32), lambda i: (i, 0)),
                  pl.BlockSpec((32, 32), lambda i: (0, 0))],
        out_specs=pl.BlockSpec((nfb, 32), lambda i: (i, 0)),
        out_shape=jax.ShapeDtypeStruct((rows, 32), jnp.float32),
        compiler_params=pltpu.CompilerParams(
            dimension_semantics=("parallel",)),
    )
    return feat(catf, w0p)


def _stage_gather(idxf, ftab):
    bt = idxf.shape[0]
    f32, i32 = jnp.float32, jnp.int32
    gather = functools.partial(
        pl.kernel,
        mesh=plsc.VectorSubcoreMesh(core_axis_name="c", subcore_axis_name="s"),
        out_type=jax.ShapeDtypeStruct((bt, 32), f32),
        scratch_types=[pltpu.VMEM((bt // (_SC_NC * _SC_NS),), i32),
                       pltpu.VMEM((2, _SC_CH, 32), f32),
                       pltpu.SemaphoreType.DMA,
                       pltpu.SemaphoreType.DMA],
        compiler_params=pltpu.CompilerParams(use_tc_tiling_on_sc=False),
    )(_sc_gather_kernel)
    return gather(idxf, ftab)


def _stage_mlp(grows, nxp, w0x, b0, W1, b1, W2, b2):
    rows = nxp.shape[0]
    K = _NSAMPLE
    mlp = pl.pallas_call(
        _mlp_kernel,
        grid=(rows // 128,),
        in_specs=[
            pl.BlockSpec((128 * K, 32), lambda i: (i, 0)),
            pl.BlockSpec((128, 8), lambda i: (i, 0)),
            pl.BlockSpec((8, 32), lambda i: (0, 0)),
            pl.BlockSpec((1, 32), lambda i: (0, 0)),
            pl.BlockSpec((32, 32), lambda i: (0, 0)),
            pl.BlockSpec((1, 32), lambda i: (0, 0)),
            pl.BlockSpec((32, 64), lambda i: (0, 0)),
            pl.BlockSpec((1, 64), lambda i: (0, 0)),
        ],
        out_specs=pl.BlockSpec((128, 64), lambda i: (i, 0)),
        out_shape=jax.ShapeDtypeStruct((rows, 64), jnp.float32),
        compiler_params=pltpu.CompilerParams(
            dimension_semantics=("parallel",)),
    )
    return mlp(grows, nxp, w0x, b0.reshape(1, 32), W1, b1.reshape(1, 32),
               W2, b2.reshape(1, 64))


def kernel(xyz, points, W0, b0, W1, b1, W2, b2):
    B, N, _ = xyz.shape
    C = points.shape[-1]
    S, K = _NPOINT, _NSAMPLE
    f32 = jnp.float32

    xs = xyz.transpose(0, 2, 1).reshape(B, 3, _NR, _NL)
    new_xyz = _stage_fps(xs)
    idx4, idxf4 = _stage_ballq(xs, new_xyz)
    idx = idx4.reshape(B, S, K)

    cat = jnp.concatenate(
        [xyz, points, jnp.zeros((B, N, 32 - 3 - C), f32)], axis=-1)
    catf = cat.reshape(B * N, 32)
    w0p = jnp.concatenate([W0, jnp.zeros((32 - 3 - C, 32), f32)], axis=0)
    ftab = _stage_feat(catf, w0p)

    grows = _stage_gather(idxf4.reshape(B * S * K), ftab)

    nxp = jnp.concatenate([new_xyz, jnp.zeros((B, S, 5), f32)],
                          axis=-1).reshape(B * S, 8)
    w0x = jnp.concatenate([W0[:3], jnp.zeros((5, 32), f32)], axis=0)
    new_points = _stage_mlp(grows, nxp, w0x, b0, W1, b1, W2, b2).reshape(
        B, S, 64)

    return new_xyz, new_points, idx
